# initial kernel scaffold (unmeasured)
import jax
import jax.numpy as jnp
from jax import lax
from jax.experimental import pallas as pl
from jax.experimental.pallas import tpu as pltpu

NZ = 4
M = 2048
D = 2048
CH = M // NZ


def kernel(partial, resid, gamma):
    gamma2 = gamma.reshape(1, D)

    def body(x_ref, resid_ref, gamma_ref, out_ref,
             rs_recv, ag_recv, rs_send_buf, ag_send_buf,
             rs_send_sems, rs_recv_sems, ag_send_sems, ag_recv_sems):
        my_x = lax.axis_index("x")
        my_y = lax.axis_index("y")
        my_z = lax.axis_index("z")
        right = lax.rem(my_z + 1, NZ)
        left = lax.rem(my_z + NZ - 1, NZ)

        barrier_sem = pltpu.get_barrier_semaphore()
        for nbr in (left, right):
            pl.semaphore_signal(
                barrier_sem, inc=1,
                device_id=(my_x, my_y, nbr),
                device_id_type=pl.DeviceIdType.MESH,
            )
        pl.semaphore_wait(barrier_sem, 2)

        def my_chunk_f32(c):
            return x_ref[0, pl.ds(c * CH, CH), :]

        for s in range(NZ - 1):
            c_send = lax.rem(my_z - s + NZ, NZ)
            if s == 0:
                rs_send_buf[...] = my_chunk_f32(c_send).astype(jnp.bfloat16)
            else:
                rs_send_buf[...] = (
                    rs_recv[s - 1] + my_chunk_f32(c_send).astype(jnp.bfloat16)
                )
            rdma = pltpu.make_async_remote_copy(
                src_ref=rs_send_buf,
                dst_ref=rs_recv.at[s],
                send_sem=rs_send_sems.at[s],
                recv_sem=rs_recv_sems.at[s],
                device_id=(my_x, my_y, right),
                device_id_type=pl.DeviceIdType.MESH,
            )
            rdma.start()
            rdma.wait()

        q = lax.rem(my_z + 1, NZ)
        y = (
            rs_recv[NZ - 2].astype(jnp.float32)
            + my_chunk_f32(q)
            + resid_ref[pl.ds(q * CH, CH), :]
        )
        rms = jnp.sqrt(jnp.mean(y * y, axis=-1, keepdims=True) + 1e-6)
        norm = (y / rms) * gamma_ref[:, :]
        out_ref[pl.ds(q * CH, CH), :] = norm
        ag_send_buf[...] = norm.astype(jnp.bfloat16)

        for h in range(NZ - 1):
            src = ag_send_buf if h == 0 else ag_recv.at[h - 1]
            rdma = pltpu.make_async_remote_copy(
                src_ref=src,
                dst_ref=ag_recv.at[h],
                send_sem=ag_send_sems.at[h],
                recv_sem=ag_recv_sems.at[h],
                device_id=(my_x, my_y, right),
                device_id_type=pl.DeviceIdType.MESH,
            )
            rdma.start()
            rdma.wait()
            c_recv = lax.rem(my_z - h + NZ, NZ)
            out_ref[pl.ds(c_recv * CH, CH), :] = ag_recv[h].astype(jnp.float32)

    return pl.pallas_call(
        body,
        out_shape=jax.ShapeDtypeStruct((M, D), jnp.float32),
        in_specs=[
            pl.BlockSpec(memory_space=pltpu.VMEM),
            pl.BlockSpec(memory_space=pltpu.VMEM),
            pl.BlockSpec(memory_space=pltpu.VMEM),
        ],
        out_specs=pl.BlockSpec(memory_space=pltpu.VMEM),
        scratch_shapes=[
            pltpu.VMEM((NZ - 1, CH, D), jnp.bfloat16),
            pltpu.VMEM((NZ - 1, CH, D), jnp.bfloat16),
            pltpu.VMEM((CH, D), jnp.bfloat16),
            pltpu.VMEM((CH, D), jnp.bfloat16),
            pltpu.SemaphoreType.DMA((NZ - 1,)),
            pltpu.SemaphoreType.DMA((NZ - 1,)),
            pltpu.SemaphoreType.DMA((NZ - 1,)),
            pltpu.SemaphoreType.DMA((NZ - 1,)),
        ],
        compiler_params=pltpu.CompilerParams(
            collective_id=0,
            vmem_limit_bytes=128 * 1024 * 1024,
        ),
    )(partial, resid, gamma2)


# baseline (device time: 177339 ns/iter reference)
import jax
import jax.numpy as jnp
from jax import lax
from jax.experimental import pallas as pl
from jax.experimental.pallas import tpu as pltpu

NZ = 4
M = 2048
D = 2048
CH = M // NZ


def kernel(partial, resid, gamma):
    gamma2 = gamma.reshape(1, D)

    def body(x_ref, resid_hbm, gamma_ref, out_ref,
             rs_recv, ag_recv, rs_send_buf, ag_send_buf, resid_chunk,
             rs_send_sems, rs_recv_sems, ag_send_sems, ag_recv_sems,
             local_sem):
        my_x = lax.axis_index("x")
        my_y = lax.axis_index("y")
        my_z = lax.axis_index("z")
        right = lax.rem(my_z + 1, NZ)
        left = lax.rem(my_z + NZ - 1, NZ)

        barrier_sem = pltpu.get_barrier_semaphore()
        for nbr in (left, right):
            pl.semaphore_signal(
                barrier_sem, inc=1,
                device_id=(my_x, my_y, nbr),
                device_id_type=pl.DeviceIdType.MESH,
            )
        pl.semaphore_wait(barrier_sem, 2)

        q = lax.rem(my_z + 1, NZ)
        resid_copy = pltpu.make_async_copy(
            resid_hbm.at[pl.ds(q * CH, CH), :], resid_chunk, local_sem
        )
        resid_copy.start()

        def my_chunk_f32(c):
            return x_ref[0, pl.ds(c * CH, CH), :]

        for s in range(NZ - 1):
            c_send = lax.rem(my_z - s + NZ, NZ)
            if s == 0:
                rs_send_buf[...] = my_chunk_f32(c_send).astype(jnp.bfloat16)
            else:
                rs_send_buf[...] = (
                    rs_recv[s - 1] + my_chunk_f32(c_send).astype(jnp.bfloat16)
                )
            rdma = pltpu.make_async_remote_copy(
                src_ref=rs_send_buf,
                dst_ref=rs_recv.at[s],
                send_sem=rs_send_sems.at[s],
                recv_sem=rs_recv_sems.at[s],
                device_id=(my_x, my_y, right),
                device_id_type=pl.DeviceIdType.MESH,
            )
            rdma.start()
            rdma.wait()

        resid_copy.wait()
        y = (
            rs_recv[NZ - 2].astype(jnp.float32)
            + my_chunk_f32(q)
            + resid_chunk[...]
        )
        rms = jnp.sqrt(jnp.mean(y * y, axis=-1, keepdims=True) + 1e-6)
        norm = (y / rms) * gamma_ref[:, :]
        out_ref[pl.ds(q * CH, CH), :] = norm
        ag_send_buf[...] = norm.astype(jnp.bfloat16)

        for h in range(NZ - 1):
            src = ag_send_buf if h == 0 else ag_recv.at[h - 1]
            rdma = pltpu.make_async_remote_copy(
                src_ref=src,
                dst_ref=ag_recv.at[h],
                send_sem=ag_send_sems.at[h],
                recv_sem=ag_recv_sems.at[h],
                device_id=(my_x, my_y, right),
                device_id_type=pl.DeviceIdType.MESH,
            )
            rdma.start()
            rdma.wait()
            c_recv = lax.rem(my_z - h + NZ, NZ)
            out_ref[pl.ds(c_recv * CH, CH), :] = ag_recv[h].astype(jnp.float32)

    return pl.pallas_call(
        body,
        out_shape=jax.ShapeDtypeStruct((M, D), jnp.float32),
        in_specs=[
            pl.BlockSpec(memory_space=pltpu.VMEM),
            pl.BlockSpec(memory_space=pl.ANY),
            pl.BlockSpec(memory_space=pltpu.VMEM),
        ],
        out_specs=pl.BlockSpec(memory_space=pltpu.VMEM),
        scratch_shapes=[
            pltpu.VMEM((NZ - 1, CH, D), jnp.bfloat16),
            pltpu.VMEM((NZ - 1, CH, D), jnp.bfloat16),
            pltpu.VMEM((CH, D), jnp.bfloat16),
            pltpu.VMEM((CH, D), jnp.bfloat16),
            pltpu.VMEM((CH, D), jnp.float32),
            pltpu.SemaphoreType.DMA((NZ - 1,)),
            pltpu.SemaphoreType.DMA((NZ - 1,)),
            pltpu.SemaphoreType.DMA((NZ - 1,)),
            pltpu.SemaphoreType.DMA((NZ - 1,)),
            pltpu.SemaphoreType.DMA,
        ],
        compiler_params=pltpu.CompilerParams(
            collective_id=0,
            vmem_limit_bytes=128 * 1024 * 1024,
        ),
    )(partial, resid, gamma2)


# device time: 110639 ns/iter; 1.6029x vs baseline; 1.6029x over previous
import jax
import jax.numpy as jnp
from jax import lax
from jax.experimental import pallas as pl
from jax.experimental.pallas import tpu as pltpu

M = 2048
D = 2048
RB = 256


def kernel(partial, resid, gamma):
    gamma2 = gamma.reshape(1, D)

    def body(x_ref, resid_hbm, gamma_ref, out_ref,
             blk_norm, a1_send, a1_recv, acc1_buf, a2_recv, resid_chunk,
             cw_recv, ccw_recv,
             a_send_sems, a_recv_sems,
             cw_send_sems, cw_recv_sems, ccw_send_sems, ccw_recv_sems,
             local_sem):
        my_x = lax.axis_index("x")
        my_y = lax.axis_index("y")
        my_z = lax.axis_index("z")
        b0 = my_z & 1
        b1 = (my_z >> 1) & 1
        k = my_x * 4 + my_y
        blk0 = k * RB
        g64 = blk0 + b0 * 128 + b1 * 64

        m = jnp.where(my_x == 0, my_y, 7 - my_y)

        def cyc_xy(c):
            cx = jnp.where(c >= 4, 1, 0)
            cy = jnp.where(c >= 4, 7 - c, c)
            return cx, cy

        cw_x, cw_y = cyc_xy(lax.rem(m + 1, 8))
        ccw_x, ccw_y = cyc_xy(lax.rem(m + 7, 8))

        barrier_sem = pltpu.get_barrier_semaphore()
        partners = [
            (my_x, my_y, my_z ^ 1),
            (my_x, my_y, my_z ^ 2),
            (cw_x, cw_y, my_z),
            (ccw_x, ccw_y, my_z),
        ]
        for p in partners:
            pl.semaphore_signal(
                barrier_sem, inc=1, device_id=p,
                device_id_type=pl.DeviceIdType.MESH,
            )
        pl.semaphore_wait(barrier_sem, 4)

        resid_copy = pltpu.make_async_copy(
            resid_hbm.at[pl.ds(g64, 64), :], resid_chunk, local_sem
        )
        resid_copy.start()

        def exchange(idx, src, dst, partner_z):
            rdma = pltpu.make_async_remote_copy(
                src_ref=src, dst_ref=dst,
                send_sem=a_send_sems.at[idx], recv_sem=a_recv_sems.at[idx],
                device_id=(my_x, my_y, partner_z),
                device_id_type=pl.DeviceIdType.MESH,
            )
            rdma.start()
            rdma.wait()

        a1_send[...] = x_ref[0, pl.ds(blk0 + (1 - b0) * 128, 128), :].astype(
            jnp.bfloat16
        )
        exchange(0, a1_send, a1_recv, my_z ^ 1)
        acc1_buf[...] = (
            a1_recv[...]
            + x_ref[0, pl.ds(blk0 + b0 * 128, 128), :].astype(jnp.bfloat16)
        )

        exchange(1, acc1_buf.at[pl.ds((1 - b1) * 64, 64), :], a2_recv, my_z ^ 2)

        resid_copy.wait()
        y = (
            (a2_recv[...] + acc1_buf[pl.ds(b1 * 64, 64), :]).astype(jnp.float32)
            + resid_chunk[...]
        )
        rms = jnp.sqrt(jnp.mean(y * y, axis=-1, keepdims=True) + 1e-6)
        norm = (y / rms) * gamma_ref[:, :]
        out_ref[pl.ds(g64, 64), :] = norm
        off64 = b0 * 128 + b1 * 64
        blk_norm[pl.ds(off64, 64), :] = norm.astype(jnp.bfloat16)

        exchange(
            2, blk_norm.at[pl.ds(off64, 64), :],
            blk_norm.at[pl.ds(off64, 64), :], my_z ^ 2,
        )
        out_ref[pl.ds(blk0 + b0 * 128 + (1 - b1) * 64, 64), :] = blk_norm[
            pl.ds(b0 * 128 + (1 - b1) * 64, 64), :
        ].astype(jnp.float32)
        exchange(
            3, blk_norm.at[pl.ds(b0 * 128, 128), :],
            blk_norm.at[pl.ds(b0 * 128, 128), :], my_z ^ 1,
        )
        out_ref[pl.ds(blk0 + (1 - b0) * 128, 128), :] = blk_norm[
            pl.ds((1 - b0) * 128, 128), :
        ].astype(jnp.float32)

        def store_piece(c, buf):
            cx, cy = cyc_xy(c)
            ko = cx * 4 + cy
            out_ref[pl.ds(ko * RB, RB), :] = buf.astype(jnp.float32)

        for h in range(4):
            cw = pltpu.make_async_remote_copy(
                src_ref=blk_norm if h == 0 else cw_recv.at[h - 1],
                dst_ref=cw_recv.at[h],
                send_sem=cw_send_sems.at[h], recv_sem=cw_recv_sems.at[h],
                device_id=(cw_x, cw_y, my_z),
                device_id_type=pl.DeviceIdType.MESH,
            )
            cw.start()
            if h < 3:
                ccw = pltpu.make_async_remote_copy(
                    src_ref=blk_norm if h == 0 else ccw_recv.at[h - 1],
                    dst_ref=ccw_recv.at[h],
                    send_sem=ccw_send_sems.at[h], recv_sem=ccw_recv_sems.at[h],
                    device_id=(ccw_x, ccw_y, my_z),
                    device_id_type=pl.DeviceIdType.MESH,
                )
                ccw.start()
            cw.wait()
            store_piece(lax.rem(m + 7 - h, 8), cw_recv[h])
            if h < 3:
                ccw.wait()
                store_piece(lax.rem(m + 1 + h, 8), ccw_recv[h])

    return pl.pallas_call(
        body,
        out_shape=jax.ShapeDtypeStruct((M, D), jnp.float32),
        in_specs=[
            pl.BlockSpec(memory_space=pltpu.VMEM),
            pl.BlockSpec(memory_space=pl.ANY),
            pl.BlockSpec(memory_space=pltpu.VMEM),
        ],
        out_specs=pl.BlockSpec(memory_space=pltpu.VMEM),
        scratch_shapes=[
            pltpu.VMEM((RB, D), jnp.bfloat16),
            pltpu.VMEM((128, D), jnp.bfloat16),
            pltpu.VMEM((128, D), jnp.bfloat16),
            pltpu.VMEM((128, D), jnp.bfloat16),
            pltpu.VMEM((64, D), jnp.bfloat16),
            pltpu.VMEM((64, D), jnp.float32),
            pltpu.VMEM((4, RB, D), jnp.bfloat16),
            pltpu.VMEM((3, RB, D), jnp.bfloat16),
            pltpu.SemaphoreType.DMA((4,)),
            pltpu.SemaphoreType.DMA((4,)),
            pltpu.SemaphoreType.DMA((4,)),
            pltpu.SemaphoreType.DMA((4,)),
            pltpu.SemaphoreType.DMA((3,)),
            pltpu.SemaphoreType.DMA((3,)),
            pltpu.SemaphoreType.DMA,
        ],
        compiler_params=pltpu.CompilerParams(
            collective_id=0,
            vmem_limit_bytes=128 * 1024 * 1024,
        ),
    )(partial, resid, gamma2)


# device time: 109494 ns/iter; 1.6196x vs baseline; 1.0105x over previous
import jax
import jax.numpy as jnp
from jax import lax
from jax.experimental import pallas as pl
from jax.experimental.pallas import tpu as pltpu

M = 2048
D = 2048
RB = 256


def kernel(partial, resid, gamma):
    gamma2 = gamma.reshape(1, D)

    def body(x_ref, resid_hbm, gamma_ref, out_ref,
             blk_norm, a1_send, a1_recv, acc1_buf, a2_recv, resid_chunk,
             cw_recv, ccw_recv,
             a_send_sems, a_recv_sems,
             cw_send_sems, cw_recv_sems, ccw_send_sems, ccw_recv_sems,
             local_sem):
        my_x = lax.axis_index("x")
        my_y = lax.axis_index("y")
        my_z = lax.axis_index("z")
        b0 = my_z & 1
        b1 = (my_z >> 1) & 1
        k = my_x * 4 + my_y
        blk0 = k * RB
        g64 = blk0 + b0 * 128 + b1 * 64

        m = jnp.where(my_x == 0, my_y, 7 - my_y)

        def cyc_xy(c):
            cx = jnp.where(c >= 4, 1, 0)
            cy = jnp.where(c >= 4, 7 - c, c)
            return cx, cy

        cw_x, cw_y = cyc_xy(lax.rem(m + 1, 8))
        ccw_x, ccw_y = cyc_xy(lax.rem(m + 7, 8))

        barrier_sem = pltpu.get_barrier_semaphore()
        partners = [
            (my_x, my_y, my_z ^ 1),
            (my_x, my_y, my_z ^ 2),
            (cw_x, cw_y, my_z),
            (ccw_x, ccw_y, my_z),
        ]
        for p in partners:
            pl.semaphore_signal(
                barrier_sem, inc=1, device_id=p,
                device_id_type=pl.DeviceIdType.MESH,
            )
        pl.semaphore_wait(barrier_sem, 4)

        resid_copy = pltpu.make_async_copy(
            resid_hbm.at[pl.ds(g64, 64), :], resid_chunk, local_sem
        )
        resid_copy.start()

        def exchange(idx, src, dst, partner_z):
            rdma = pltpu.make_async_remote_copy(
                src_ref=src, dst_ref=dst,
                send_sem=a_send_sems.at[idx], recv_sem=a_recv_sems.at[idx],
                device_id=(my_x, my_y, partner_z),
                device_id_type=pl.DeviceIdType.MESH,
            )
            rdma.start()
            rdma.wait()

        a1_send[...] = x_ref[0, pl.ds(blk0 + (1 - b0) * 128, 128), :].astype(
            jnp.bfloat16
        )
        exchange(0, a1_send, a1_recv, my_z ^ 1)
        acc1_buf[...] = (
            a1_recv[...]
            + x_ref[0, pl.ds(blk0 + b0 * 128, 128), :].astype(jnp.bfloat16)
        )

        exchange(1, acc1_buf.at[pl.ds((1 - b1) * 64, 64), :], a2_recv, my_z ^ 2)

        resid_copy.wait()
        y = (
            (a2_recv[...] + acc1_buf[pl.ds(b1 * 64, 64), :]).astype(jnp.float32)
            + resid_chunk[...]
        )
        rms = jnp.sqrt(jnp.mean(y * y, axis=-1, keepdims=True) + 1e-6)
        norm = (y / rms) * gamma_ref[:, :]
        off64 = b0 * 128 + b1 * 64
        blk_norm[pl.ds(off64, 64), :] = norm.astype(jnp.bfloat16)

        def start_exchange(idx, src, dst, partner_z):
            rdma = pltpu.make_async_remote_copy(
                src_ref=src, dst_ref=dst,
                send_sem=a_send_sems.at[idx], recv_sem=a_recv_sems.at[idx],
                device_id=(my_x, my_y, partner_z),
                device_id_type=pl.DeviceIdType.MESH,
            )
            rdma.start()
            return rdma

        ag2 = start_exchange(
            2, blk_norm.at[pl.ds(off64, 64), :],
            blk_norm.at[pl.ds(off64, 64), :], my_z ^ 2,
        )
        out_ref[pl.ds(g64, 64), :] = norm
        ag2.wait()
        ag1 = start_exchange(
            3, blk_norm.at[pl.ds(b0 * 128, 128), :],
            blk_norm.at[pl.ds(b0 * 128, 128), :], my_z ^ 1,
        )
        out_ref[pl.ds(blk0 + b0 * 128 + (1 - b1) * 64, 64), :] = blk_norm[
            pl.ds(b0 * 128 + (1 - b1) * 64, 64), :
        ].astype(jnp.float32)
        ag1.wait()

        def start_hop(h):
            cw = pltpu.make_async_remote_copy(
                src_ref=blk_norm if h == 0 else cw_recv.at[h - 1],
                dst_ref=cw_recv.at[h],
                send_sem=cw_send_sems.at[h], recv_sem=cw_recv_sems.at[h],
                device_id=(cw_x, cw_y, my_z),
                device_id_type=pl.DeviceIdType.MESH,
            )
            cw.start()
            ccw = None
            if h < 3:
                ccw = pltpu.make_async_remote_copy(
                    src_ref=blk_norm if h == 0 else ccw_recv.at[h - 1],
                    dst_ref=ccw_recv.at[h],
                    send_sem=ccw_send_sems.at[h], recv_sem=ccw_recv_sems.at[h],
                    device_id=(ccw_x, ccw_y, my_z),
                    device_id_type=pl.DeviceIdType.MESH,
                )
                ccw.start()
            return cw, ccw

        def store_piece(c, buf):
            cx, cy = cyc_xy(c)
            ko = cx * 4 + cy
            out_ref[pl.ds(ko * RB, RB), :] = buf.astype(jnp.float32)

        cw, ccw = start_hop(0)
        out_ref[pl.ds(blk0 + (1 - b0) * 128, 128), :] = blk_norm[
            pl.ds((1 - b0) * 128, 128), :
        ].astype(jnp.float32)
        for h in range(4):
            cw.wait()
            if ccw is not None:
                ccw.wait()
            if h < 3:
                nxt_cw, nxt_ccw = start_hop(h + 1)
            store_piece(lax.rem(m + 7 - h, 8), cw_recv[h])
            if h < 3:
                store_piece(lax.rem(m + 1 + h, 8), ccw_recv[h])
                cw, ccw = nxt_cw, nxt_ccw

    return pl.pallas_call(
        body,
        out_shape=jax.ShapeDtypeStruct((M, D), jnp.float32),
        in_specs=[
            pl.BlockSpec(memory_space=pltpu.VMEM),
            pl.BlockSpec(memory_space=pl.ANY),
            pl.BlockSpec(memory_space=pltpu.VMEM),
        ],
        out_specs=pl.BlockSpec(memory_space=pltpu.VMEM),
        scratch_shapes=[
            pltpu.VMEM((RB, D), jnp.bfloat16),
            pltpu.VMEM((128, D), jnp.bfloat16),
            pltpu.VMEM((128, D), jnp.bfloat16),
            pltpu.VMEM((128, D), jnp.bfloat16),
            pltpu.VMEM((64, D), jnp.bfloat16),
            pltpu.VMEM((64, D), jnp.float32),
            pltpu.VMEM((4, RB, D), jnp.bfloat16),
            pltpu.VMEM((3, RB, D), jnp.bfloat16),
            pltpu.SemaphoreType.DMA((4,)),
            pltpu.SemaphoreType.DMA((4,)),
            pltpu.SemaphoreType.DMA((4,)),
            pltpu.SemaphoreType.DMA((4,)),
            pltpu.SemaphoreType.DMA((3,)),
            pltpu.SemaphoreType.DMA((3,)),
            pltpu.SemaphoreType.DMA,
        ],
        compiler_params=pltpu.CompilerParams(
            collective_id=0,
            vmem_limit_bytes=128 * 1024 * 1024,
        ),
    )(partial, resid, gamma2)
